# Initial kernel scaffold; baseline (speedup 1.0000x reference)
import jax
import jax.numpy as jnp
from jax.experimental import pallas as pl


def kernel(x):
    raise NotImplementedError("write your pallas kernel here")


# speedup vs baseline: 2.4682x; 2.4682x over previous
import functools

import jax
import jax.numpy as jnp
from jax import lax
from jax.experimental import pallas as pl
from jax.experimental.pallas import tpu as pltpu

H = 128
W = 128
SY = 4
SX = 4
N = H * W
C = 256
ND = (H // SY) * (W // SX)
R_MERGE = min(N // 2, N - ND)
BN = 1024
NB = N // BN


def _scores_body(xb_ref, xd_ref, nmax_ref, nidx_ref):
    xb = xb_ref[0]
    xd = xd_ref[0]
    mb = xb / (jnp.sqrt(jnp.sum(xb * xb, axis=1, keepdims=True)) + 1e-6)
    md = xd / (jnp.sqrt(jnp.sum(xd * xd, axis=1, keepdims=True)) + 1e-6)
    s = lax.dot_general(mb, md, (((1,), (1,)), ((), ())))
    nmax = jnp.max(s, axis=1)
    iota = lax.broadcasted_iota(jnp.int32, (BN, ND), 1)
    nidx = jnp.min(jnp.where(s == nmax[:, None], iota, ND), axis=1)
    nmax_ref[0, 0] = nmax
    nidx_ref[0, 0] = nidx.astype(jnp.int32)


def _select_body(nm_ref, ni_ref, sc_ref, gi_ref):
    nm = nm_ref[0]
    ni = ni_ref[0]
    ri = lax.broadcasted_iota(jnp.int32, (H, W), 0)
    ci = lax.broadcasted_iota(jnp.int32, (H, W), 1)
    is_dst = ((ri % SY) == 0) & ((ci % SX) == 0)
    bits = lax.bitcast_convert_type(nm, jnp.int32)
    v = bits ^ (lax.shift_right_arithmetic(bits, 31) & jnp.int32(0x7FFFFFFF))
    min32 = jnp.int32(-2147483648)
    v = jnp.where(is_dst, min32, v)
    r = jnp.int32(R_MERGE)
    cnt_pos = jnp.sum((v >= 0).astype(jnp.int32))
    bucket_pos = cnt_pos >= r
    in_bucket = ((v >= 0) == bucket_pos) & jnp.logical_not(is_dst)
    key = jnp.where(in_bucket, v & jnp.int32(0x7FFFFFFF), jnp.int32(-1))
    rr = jnp.where(bucket_pos, r, r - cnt_pos)

    def body(k, prefix):
        cand = prefix | lax.shift_left(jnp.int32(1), jnp.int32(30) - k)
        cnt = jnp.sum((key >= cand).astype(jnp.int32))
        return jnp.where(cnt >= rr, cand, prefix)

    t = lax.fori_loop(0, 31, body, jnp.int32(0))
    tv = jnp.where(bucket_pos, t, t | min32)
    not_dst = jnp.logical_not(is_dst)
    gt = (v > tv) & not_dst
    eq = (v == tv) & not_dst
    cnt_gt = jnp.sum(gt.astype(jnp.int32))
    need = (r - cnt_gt).astype(jnp.float32)
    e = eq.astype(jnp.float32)
    tri_incl = (ri <= ci).astype(jnp.float32)
    incl = lax.dot_general(e, tri_incl, (((1,), (0,)), ((), ())),
                           precision=lax.Precision.HIGHEST)
    row_tot = incl[:, W - 1:W]
    tri_strict = (ri > ci).astype(jnp.float32)
    off = lax.dot_general(tri_strict, row_tot, (((1,), (0,)), ((), ())),
                          precision=lax.Precision.HIGHEST)
    rank_excl = incl - e + off
    merged = gt | (eq & (rank_excl < need))
    sc_ref[0] = jnp.where(merged, ni, jnp.int32(ND))
    dbin = (ri // SY) * (W // SX) + (ci // SX)
    gi_ref[0] = jnp.where(is_dst, dbin, jnp.where(merged, ni, jnp.int32(-1)))


def _scatter_body(xb_ref, si_ref, xd_ref, dm_ref, acc_ref, cnt_ref):
    nb = pl.program_id(1)

    @pl.when(nb == 0)
    def _():
        acc_ref[...] = jnp.zeros_like(acc_ref)
        cnt_ref[...] = jnp.zeros_like(cnt_ref)

    xb = xb_ref[0]
    si = jnp.broadcast_to(si_ref[0], (ND, BN))
    iota = lax.broadcasted_iota(jnp.int32, (ND, BN), 0)
    oht = (si == iota).astype(jnp.float32)
    acc_ref[...] += lax.dot_general(oht, xb, (((1,), (0,)), ((), ())),
                                    precision=lax.Precision.HIGHEST)
    cnt_ref[...] += lax.dot_general(oht, jnp.ones((BN, 128), jnp.float32),
                                    (((1,), (0,)), ((), ())),
                                    precision=lax.Precision.HIGHEST)

    @pl.when(nb == NB - 1)
    def _():
        dm_ref[0] = (xd_ref[0] + acc_ref[...]) / (1.0 + cnt_ref[:, 0:1])


def _gather_body(xb_ref, gi_ref, dm_ref, out_ref):
    xb = xb_ref[0]
    g = jnp.broadcast_to(gi_ref[0], (ND, BN))
    iota = lax.broadcasted_iota(jnp.int32, (ND, BN), 0)
    oht = (g == iota).astype(jnp.float32)
    gathered = lax.dot_general(oht, dm_ref[0], (((0,), (0,)), ((), ())),
                               precision=lax.Precision.HIGHEST)
    hit = lax.dot_general(oht, jnp.ones((ND, 128), jnp.float32),
                          (((0,), (0,)), ((), ())),
                          precision=lax.Precision.HIGHEST)
    out_ref[0] = xb * (1.0 - hit[:, 0:1]) + gathered


def kernel(x):
    B = x.shape[0]
    x_dst = x.reshape(B, H // SY, SY, W // SX, SX, C)[:, :, 0, :, 0, :]
    x_dst = x_dst.reshape(B, ND, C)

    nmax, nidx = pl.pallas_call(
        _scores_body,
        grid=(B, NB),
        in_specs=[
            pl.BlockSpec((1, BN, C), lambda b, nb: (b, nb, 0)),
            pl.BlockSpec((1, ND, C), lambda b, nb: (b, 0, 0)),
        ],
        out_specs=[
            pl.BlockSpec((1, 1, BN), lambda b, nb: (b * NB + nb, 0, 0)),
            pl.BlockSpec((1, 1, BN), lambda b, nb: (b * NB + nb, 0, 0)),
        ],
        out_shape=[
            jax.ShapeDtypeStruct((B * NB, 1, BN), jnp.float32),
            jax.ShapeDtypeStruct((B * NB, 1, BN), jnp.int32),
        ],
    )(x, x_dst)

    scat, gath = pl.pallas_call(
        _select_body,
        grid=(B,),
        in_specs=[
            pl.BlockSpec((1, H, W), lambda b: (b, 0, 0)),
            pl.BlockSpec((1, H, W), lambda b: (b, 0, 0)),
        ],
        out_specs=[
            pl.BlockSpec((1, H, W), lambda b: (b, 0, 0)),
            pl.BlockSpec((1, H, W), lambda b: (b, 0, 0)),
        ],
        out_shape=[
            jax.ShapeDtypeStruct((B, H, W), jnp.int32),
            jax.ShapeDtypeStruct((B, H, W), jnp.int32),
        ],
    )(nmax.reshape(B, H, W), nidx.reshape(B, H, W))

    dst_mean = pl.pallas_call(
        _scatter_body,
        grid=(B, NB),
        in_specs=[
            pl.BlockSpec((1, BN, C), lambda b, nb: (b, nb, 0)),
            pl.BlockSpec((1, 1, BN), lambda b, nb: (b * NB + nb, 0, 0)),
            pl.BlockSpec((1, ND, C), lambda b, nb: (b, 0, 0)),
        ],
        out_specs=pl.BlockSpec((1, ND, C), lambda b, nb: (b, 0, 0)),
        out_shape=jax.ShapeDtypeStruct((B, ND, C), jnp.float32),
        scratch_shapes=[
            pltpu.VMEM((ND, C), jnp.float32),
            pltpu.VMEM((ND, 128), jnp.float32),
        ],
    )(x, scat.reshape(B * NB, 1, BN), x_dst)

    out = pl.pallas_call(
        _gather_body,
        grid=(B, NB),
        in_specs=[
            pl.BlockSpec((1, BN, C), lambda b, nb: (b, nb, 0)),
            pl.BlockSpec((1, 1, BN), lambda b, nb: (b * NB + nb, 0, 0)),
            pl.BlockSpec((1, ND, C), lambda b, nb: (b, 0, 0)),
        ],
        out_specs=pl.BlockSpec((1, BN, C), lambda b, nb: (b, nb, 0)),
        out_shape=jax.ShapeDtypeStruct((B, N, C), jnp.float32),
    )(x, gath.reshape(B * NB, 1, BN), dst_mean)

    return out
